# baseline (speedup 1.0000x reference)
import functools

import jax
import jax.numpy as jnp
from jax import lax
from jax.experimental import pallas as pl
from jax.experimental.pallas import tpu as pltpu
from jax.experimental.pallas import tpu_sc as plsc

_N_NODES = 100000
_N_EDGES = 6400000
_CUTOFF = 5.0
_N_BASES = 6

_NC = 2
_NS = 16
_NW = _NC * _NS
_L = 16

_NSPLIT = 2
_NE_S = _N_EDGES // _NSPLIT
_C = 1024
_NCHUNKS = _NE_S // _C
_GROUPS = _C // _L
_IDX_B = 128
_NIDX = 2 * _C // _IDX_B
_RPC = _C // _IDX_B
_DPAD = 8
_NK = 98

_COSC = (1.00000000e+00, -5.00000000e-01, 4.16666665e-02, -1.38888849e-03,
         2.48011029e-05, -2.75271125e-07, 1.99427735e-09)
_SINC = (1.00000000e+00, -1.66666666e-01, 8.33332995e-03, -1.98407728e-04,
         2.75219338e-06, -2.38435661e-08)

_PI = 3.14159265358979
_W1 = _PI / _CUTOFF
_PREF = 0.6324555320336759


def _rsqrt(x):
    i = lax.bitcast_convert_type(x, jnp.int32)
    i = jnp.int32(0x5F3759DF) - lax.shift_right_arithmetic(i, jnp.int32(1))
    y = lax.bitcast_convert_type(i, jnp.float32)
    half = jnp.float32(0.5) * x
    for _ in range(3):
        y = y * (jnp.float32(1.5) - half * y * y)
    return y


def _sc_body(pos_hbm, send_hbm, recv_hbm, shx_hbm, shy_hbm, shz_hbm,
             len_out, e0_out, e1_out, e2_out, e3_out, e4_out, e5_out,
             ux_out, uy_out, uz_out,
             idx_v, rows_v, shx_v, shy_v, shz_v,
             len_v, e_v, u_v, sem0, sem1):
    wid = lax.axis_index("s") * _NC + lax.axis_index("c")
    sems = (sem0, sem1)

    def gather_copies(b, cid):
        base = cid * _C
        cps = []
        for j in range(_NIDX):
            cps.append((pos_hbm.at[idx_v.at[b, j]],
                        rows_v.at[b, pl.ds(j * _IDX_B, _IDX_B)]))
        cps.append((shx_hbm.at[pl.ds(base, _C)], shx_v.at[b]))
        cps.append((shy_hbm.at[pl.ds(base, _C)], shy_v.at[b]))
        cps.append((shz_hbm.at[pl.ds(base, _C)], shz_v.at[b]))
        return cps

    def start(b, k):
        cid = wid + k * _NW

        def _go():
            rbase = cid * _RPC
            pltpu.sync_copy(send_hbm.at[pl.ds(rbase, _RPC)],
                            idx_v.at[b, pl.ds(0, _RPC)])
            pltpu.sync_copy(recv_hbm.at[pl.ds(rbase, _RPC)],
                            idx_v.at[b, pl.ds(_RPC, _RPC)])
            for src, dst in gather_copies(b, cid):
                pltpu.async_copy(src, dst, sems[b])

        pl.when(cid < _NCHUNKS)(_go)

    def finish(b, k):
        cid = wid + k * _NW

        def _go():
            base = cid * _C
            for src, dst in gather_copies(b, cid):
                pltpu.make_async_copy(src, dst, sems[b]).wait()

            @plsc.parallel_loop(0, _C, _L, unroll=2)
            def group(o):
                e = lax.iota(jnp.int32, _L) + o
                er = e + _C
                c0 = jnp.zeros((_L,), jnp.int32)
                c1i = jnp.full((_L,), 1, jnp.int32)
                c2i = jnp.full((_L,), 2, jnp.int32)
                rv = rows_v.at[b]
                dx = (plsc.load_gather(rv, [er, c0])
                      - plsc.load_gather(rv, [e, c0])
                      + shx_v[b, pl.ds(o, _L)])
                dy = (plsc.load_gather(rv, [er, c1i])
                      - plsc.load_gather(rv, [e, c1i])
                      + shy_v[b, pl.ds(o, _L)])
                dz = (plsc.load_gather(rv, [er, c2i])
                      - plsc.load_gather(rv, [e, c2i])
                      + shz_v[b, pl.ds(o, _L)])
                len2 = dx * dx + dy * dy + dz * dz
                inv = _rsqrt(len2)
                length = len2 * inv
                len_v[b, pl.ds(o, _L)] = length
                u_v[b, 0, pl.ds(o, _L)] = dx * inv
                u_v[b, 1, pl.ds(o, _L)] = dy * inv
                u_v[b, 2, pl.ds(o, _L)] = dz * inv

                t = jnp.float32(_W1) * lax.min(length, jnp.float32(_CUTOFF))
                u = t - jnp.float32(_PI / 2)
                v = u * u
                s1 = jnp.float32(_COSC[6])
                for cc in (_COSC[5], _COSC[4], _COSC[3], _COSC[2], _COSC[1],
                           _COSC[0]):
                    s1 = s1 * v + jnp.float32(cc)
                cpoly = jnp.float32(_SINC[5])
                for cc in (_SINC[4], _SINC[3], _SINC[2], _SINC[1], _SINC[0]):
                    cpoly = cpoly * v + jnp.float32(cc)
                ct = -(cpoly * u)

                x = length * jnp.float32(1.0 / _CUTOFF)
                x2 = x * x
                x3 = x2 * x
                x6 = x3 * x3
                x7 = x6 * x
                x8 = x7 * x
                env = (jnp.float32(1.0) - jnp.float32(28.0) * x6
                       + jnp.float32(48.0) * x7 - jnp.float32(21.0) * x8)
                env = jnp.where(x < jnp.float32(1.0), env, jnp.float32(0.0))
                scale = jnp.float32(_PREF) * inv * env

                twoc = jnp.float32(2.0) * ct
                sprev = jnp.zeros((_L,), jnp.float32)
                scur = s1
                for n in range(_N_BASES):
                    e_v[b, n, pl.ds(o, _L)] = scur * scale
                    snext = twoc * scur - sprev
                    sprev, scur = scur, snext

            pltpu.sync_copy(len_v.at[b], len_out.at[pl.ds(base, _C)])
            for n, ref in enumerate((e0_out, e1_out, e2_out, e3_out, e4_out,
                                     e5_out)):
                pltpu.sync_copy(e_v.at[b, n], ref.at[pl.ds(base, _C)])
            pltpu.sync_copy(u_v.at[b, 0], ux_out.at[pl.ds(base, _C)])
            pltpu.sync_copy(u_v.at[b, 1], uy_out.at[pl.ds(base, _C)])
            pltpu.sync_copy(u_v.at[b, 2], uz_out.at[pl.ds(base, _C)])

        pl.when(cid < _NCHUNKS)(_go)

    start(0, 0)

    def pair(i, carry):
        k = 2 * i
        start(1, k + 1)
        finish(0, k)
        start(0, k + 2)
        finish(1, k + 1)
        return carry

    lax.fori_loop(0, _NK // 2, pair, 0)


_E1 = jax.ShapeDtypeStruct((_NE_S,), jnp.float32)
_sc_kernel = functools.partial(
    pl.kernel,
    mesh=plsc.VectorSubcoreMesh(core_axis_name="c", subcore_axis_name="s"),
    out_type=(_E1,) * 10,
    scratch_types=[
        pltpu.VMEM((2, 2 * _C // _IDX_B, _IDX_B), jnp.int32),
        pltpu.VMEM((2, 2 * _C, _DPAD), jnp.float32),
        pltpu.VMEM((2, _C), jnp.float32),
        pltpu.VMEM((2, _C), jnp.float32),
        pltpu.VMEM((2, _C), jnp.float32),
        pltpu.VMEM((2, _C), jnp.float32),
        pltpu.VMEM((2, _N_BASES, _C), jnp.float32),
        pltpu.VMEM((2, 3, _C), jnp.float32),
        pltpu.SemaphoreType.DMA,
        pltpu.SemaphoreType.DMA,
    ],
    compiler_params=pltpu.CompilerParams(
        needs_layout_passes=False, use_tc_tiling_on_sc=False),
)(_sc_body)


@jax.jit
def kernel(positions, edge_index, shifts):
    pos_pad = jnp.concatenate(
        [positions, jnp.zeros((_N_NODES, _DPAD - 3), jnp.float32)], axis=1)
    lens, embs, units = [], [], []
    for s in range(_NSPLIT):
        lo = s * _NE_S
        sender = lax.slice(edge_index, (0, lo), (1, lo + _NE_S)).reshape(
            _NE_S // _IDX_B, _IDX_B)
        receiver = lax.slice(edge_index, (1, lo), (2, lo + _NE_S)).reshape(
            _NE_S // _IDX_B, _IDX_B)
        shs = lax.slice(shifts, (lo, 0), (lo + _NE_S, 3))
        outs = _sc_kernel(pos_pad, sender, receiver,
                          shs[:, 0], shs[:, 1], shs[:, 2])
        lengths, e0, e1, e2, e3, e4, e5, ux, uy, uz = outs
        lens.append(lengths.reshape(_NE_S, 1))
        embs.append(jnp.stack([e0, e1, e2, e3, e4, e5], axis=1))
        units.append(jnp.stack([ux, uy, uz], axis=1))
    return (jnp.concatenate(lens, axis=0),
            jnp.concatenate(embs, axis=0),
            jnp.concatenate(units, axis=0))


# speedup vs baseline: 1.9095x; 1.9095x over previous
import functools

import jax
import jax.numpy as jnp
from jax import lax
from jax.experimental import pallas as pl
from jax.experimental.pallas import tpu as pltpu
from jax.experimental.pallas import tpu_sc as plsc

_N_NODES = 100000
_N_EDGES = 6400000
_CUTOFF = 5.0
_N_BASES = 6

_NC = 2
_NS = 16
_NW = _NC * _NS
_L = 16

_C = 1024
_NCHUNKS = _N_EDGES // _C
_GROUPS = _C // _L
_IDX_B = 128
_NIDX = 2 * _C // _IDX_B
_RPC = _C // _IDX_B
_BPC = _C // _IDX_B
_DPAD = 8
_NK = 196
_NBLK = _N_EDGES // _IDX_B

_COSC = (1.00000000e+00, -5.00000000e-01, 4.16666665e-02, -1.38888849e-03,
         2.48011029e-05, -2.75271125e-07, 1.99427735e-09)
_SINC = (1.00000000e+00, -1.66666666e-01, 8.33332995e-03, -1.98407728e-04,
         2.75219338e-06, -2.38435661e-08)

_PI = 3.14159265358979
_W1 = _PI / _CUTOFF
_PREF = 0.6324555320336759


def _rsqrt(x):
    i = lax.bitcast_convert_type(x, jnp.int32)
    i = jnp.int32(0x5F3759DF) - lax.shift_right_arithmetic(i, jnp.int32(1))
    y = lax.bitcast_convert_type(i, jnp.float32)
    half = jnp.float32(0.5) * x
    for _ in range(3):
        y = y * (jnp.float32(1.5) - half * y * y)
    return y


def _sc_body(pos_hbm, send_hbm, recv_hbm, shx_hbm, shy_hbm, shz_hbm,
             len_out, emb_out, unit_out,
             idx_v, rows_v, shx_v, shy_v, shz_v,
             len_v, e_v, u_v, sem0, sem1):
    wid = lax.axis_index("s") * _NC + lax.axis_index("c")
    sems = (sem0, sem1)

    def gather_copies(b, cid):
        base = cid * _C
        cps = []
        for j in range(_NIDX):
            cps.append((pos_hbm.at[idx_v.at[b, j]],
                        rows_v.at[b, pl.ds(j * _IDX_B, _IDX_B)]))
        cps.append((shx_hbm.at[pl.ds(base, _C)], shx_v.at[b]))
        cps.append((shy_hbm.at[pl.ds(base, _C)], shy_v.at[b]))
        cps.append((shz_hbm.at[pl.ds(base, _C)], shz_v.at[b]))
        return cps

    def start(b, k):
        cid = wid + k * _NW

        def _go():
            rbase = cid * _RPC
            pltpu.sync_copy(send_hbm.at[pl.ds(rbase, _RPC)],
                            idx_v.at[b, pl.ds(0, _RPC)])
            pltpu.sync_copy(recv_hbm.at[pl.ds(rbase, _RPC)],
                            idx_v.at[b, pl.ds(_RPC, _RPC)])
            for src, dst in gather_copies(b, cid):
                pltpu.async_copy(src, dst, sems[b])

        pl.when(cid < _NCHUNKS)(_go)

    def finish(b, k):
        cid = wid + k * _NW

        def _go():
            base = cid * _C
            for src, dst in gather_copies(b, cid):
                pltpu.make_async_copy(src, dst, sems[b]).wait()

            @plsc.parallel_loop(0, _C, _L, unroll=2)
            def group(o):
                blk = lax.div(o, jnp.int32(_IDX_B))
                ol = o - blk * _IDX_B
                e = lax.iota(jnp.int32, _L) + o
                er = e + _C
                c0 = jnp.zeros((_L,), jnp.int32)
                c1i = jnp.full((_L,), 1, jnp.int32)
                c2i = jnp.full((_L,), 2, jnp.int32)
                rv = rows_v.at[b]
                dx = (plsc.load_gather(rv, [er, c0])
                      - plsc.load_gather(rv, [e, c0])
                      + shx_v[b, pl.ds(o, _L)])
                dy = (plsc.load_gather(rv, [er, c1i])
                      - plsc.load_gather(rv, [e, c1i])
                      + shy_v[b, pl.ds(o, _L)])
                dz = (plsc.load_gather(rv, [er, c2i])
                      - plsc.load_gather(rv, [e, c2i])
                      + shz_v[b, pl.ds(o, _L)])
                len2 = dx * dx + dy * dy + dz * dz
                inv = _rsqrt(len2)
                length = len2 * inv
                len_v[b, pl.ds(o, _L)] = length
                u_v[b, blk, 0, pl.ds(ol, _L)] = dx * inv
                u_v[b, blk, 1, pl.ds(ol, _L)] = dy * inv
                u_v[b, blk, 2, pl.ds(ol, _L)] = dz * inv

                t = jnp.float32(_W1) * lax.min(length, jnp.float32(_CUTOFF))
                u = t - jnp.float32(_PI / 2)
                v = u * u
                s1 = jnp.float32(_COSC[6])
                for cc in (_COSC[5], _COSC[4], _COSC[3], _COSC[2], _COSC[1],
                           _COSC[0]):
                    s1 = s1 * v + jnp.float32(cc)
                cpoly = jnp.float32(_SINC[5])
                for cc in (_SINC[4], _SINC[3], _SINC[2], _SINC[1], _SINC[0]):
                    cpoly = cpoly * v + jnp.float32(cc)
                ct = -(cpoly * u)

                x = length * jnp.float32(1.0 / _CUTOFF)
                x2 = x * x
                x3 = x2 * x
                x6 = x3 * x3
                x7 = x6 * x
                x8 = x7 * x
                env = (jnp.float32(1.0) - jnp.float32(28.0) * x6
                       + jnp.float32(48.0) * x7 - jnp.float32(21.0) * x8)
                env = jnp.where(x < jnp.float32(1.0), env, jnp.float32(0.0))
                scale = jnp.float32(_PREF) * inv * env

                twoc = jnp.float32(2.0) * ct
                sprev = jnp.zeros((_L,), jnp.float32)
                scur = s1
                for n in range(_N_BASES):
                    e_v[b, blk, n, pl.ds(ol, _L)] = scur * scale
                    snext = twoc * scur - sprev
                    sprev, scur = scur, snext

            bbase = cid * _BPC
            pltpu.sync_copy(len_v.at[b], len_out.at[pl.ds(base, _C)])
            pltpu.sync_copy(e_v.at[b], emb_out.at[pl.ds(bbase, _BPC)])
            pltpu.sync_copy(u_v.at[b], unit_out.at[pl.ds(bbase, _BPC)])

        pl.when(cid < _NCHUNKS)(_go)

    start(0, 0)

    def pair(i, carry):
        k = 2 * i
        start(1, k + 1)
        finish(0, k)
        start(0, k + 2)
        finish(1, k + 1)
        return carry

    lax.fori_loop(0, _NK // 2, pair, 0)


_sc_kernel = functools.partial(
    pl.kernel,
    mesh=plsc.VectorSubcoreMesh(core_axis_name="c", subcore_axis_name="s"),
    out_type=(
        jax.ShapeDtypeStruct((_N_EDGES,), jnp.float32),
        jax.ShapeDtypeStruct((_NBLK, 8, _IDX_B), jnp.float32),
        jax.ShapeDtypeStruct((_NBLK, 4, _IDX_B), jnp.float32),
    ),
    scratch_types=[
        pltpu.VMEM((2, 2 * _C // _IDX_B, _IDX_B), jnp.int32),
        pltpu.VMEM((2, 2 * _C, _DPAD), jnp.float32),
        pltpu.VMEM((2, _C), jnp.float32),
        pltpu.VMEM((2, _C), jnp.float32),
        pltpu.VMEM((2, _C), jnp.float32),
        pltpu.VMEM((2, _C), jnp.float32),
        pltpu.VMEM((2, _BPC, 8, _IDX_B), jnp.float32),
        pltpu.VMEM((2, _BPC, 4, _IDX_B), jnp.float32),
        pltpu.SemaphoreType.DMA,
        pltpu.SemaphoreType.DMA,
    ],
    compiler_params=pltpu.CompilerParams(
        needs_layout_passes=False, use_tc_tiling_on_sc=False),
)(_sc_body)


@jax.jit
def kernel(positions, edge_index, shifts):
    pos_pad = jnp.concatenate(
        [positions, jnp.zeros((_N_NODES, _DPAD - 3), jnp.float32)], axis=1)
    sender = edge_index[0].reshape(_N_EDGES // _IDX_B, _IDX_B)
    receiver = edge_index[1].reshape(_N_EDGES // _IDX_B, _IDX_B)
    lengths, emb3, unit3 = _sc_kernel(pos_pad, sender, receiver,
                                      shifts[:, 0], shifts[:, 1],
                                      shifts[:, 2])
    emb = emb3.transpose(0, 2, 1).reshape(_N_EDGES, 8)[:, :_N_BASES]
    unit = unit3.transpose(0, 2, 1).reshape(_N_EDGES, 4)[:, :3]
    return (lengths.reshape(_N_EDGES, 1), emb, unit)


# speedup vs baseline: 2.0372x; 1.0669x over previous
import functools

import jax
import jax.numpy as jnp
from jax import lax
from jax.experimental import pallas as pl
from jax.experimental.pallas import tpu as pltpu
from jax.experimental.pallas import tpu_sc as plsc

_N_NODES = 100000
_N_EDGES = 6400000
_CUTOFF = 5.0
_N_BASES = 6

_NC = 2
_NS = 16
_NW = _NC * _NS
_L = 16

_C = 1024
_NCHUNKS = _N_EDGES // _C
_GROUPS = _C // _L
_IDX_B = 128
_NIDX = 2 * _C // _IDX_B
_RPC = _C // _IDX_B
_BPC = _C // _IDX_B
_DPAD = 8
_NK = 196
_NBLK = _N_EDGES // _IDX_B

_COSC = (1.00000000e+00, -5.00000000e-01, 4.16666665e-02, -1.38888849e-03,
         2.48011029e-05, -2.75271125e-07, 1.99427735e-09)
_SINC = (1.00000000e+00, -1.66666666e-01, 8.33332995e-03, -1.98407728e-04,
         2.75219338e-06, -2.38435661e-08)

_PI = 3.14159265358979
_W1 = _PI / _CUTOFF
_PREF = 0.6324555320336759


def _rsqrt(x):
    i = lax.bitcast_convert_type(x, jnp.int32)
    i = jnp.int32(0x5F3759DF) - lax.shift_right_arithmetic(i, jnp.int32(1))
    y = lax.bitcast_convert_type(i, jnp.float32)
    half = jnp.float32(0.5) * x
    for _ in range(3):
        y = y * (jnp.float32(1.5) - half * y * y)
    return y


def _sc_body(pos_hbm, send_hbm, recv_hbm, shx_hbm, shy_hbm, shz_hbm,
             len_out, emb_out, unit_out,
             idx_v, rows_v, shx_v, shy_v, shz_v,
             len_v, e_v, u_v, sem0, sem1, osem0, osem1):
    wid = lax.axis_index("s") * _NC + lax.axis_index("c")
    sems = (sem0, sem1)
    osems = (osem0, osem1)

    def out_copies(b, cid):
        base = cid * _C
        bbase = cid * _BPC
        return ((len_v.at[b], len_out.at[pl.ds(base, _C)]),
                (e_v.at[b], emb_out.at[pl.ds(bbase, _BPC)]),
                (u_v.at[b], unit_out.at[pl.ds(bbase, _BPC)]))

    def drain_outputs(b, cid):
        def _go():
            for src, dst in out_copies(b, cid):
                pltpu.make_async_copy(src, dst, osems[b]).wait()

        pl.when(jnp.logical_and(cid >= 0, cid < _NCHUNKS))(_go)

    def gather_copies(b, cid):
        base = cid * _C
        cps = []
        for j in range(_NIDX):
            cps.append((pos_hbm.at[idx_v.at[b, j]],
                        rows_v.at[b, pl.ds(j * _IDX_B, _IDX_B)]))
        cps.append((shx_hbm.at[pl.ds(base, _C)], shx_v.at[b]))
        cps.append((shy_hbm.at[pl.ds(base, _C)], shy_v.at[b]))
        cps.append((shz_hbm.at[pl.ds(base, _C)], shz_v.at[b]))
        return cps

    def start(b, k):
        cid = wid + k * _NW

        def _go():
            rbase = cid * _RPC
            pltpu.sync_copy(send_hbm.at[pl.ds(rbase, _RPC)],
                            idx_v.at[b, pl.ds(0, _RPC)])
            pltpu.sync_copy(recv_hbm.at[pl.ds(rbase, _RPC)],
                            idx_v.at[b, pl.ds(_RPC, _RPC)])
            for src, dst in gather_copies(b, cid):
                pltpu.async_copy(src, dst, sems[b])

        pl.when(cid < _NCHUNKS)(_go)

    def finish(b, k):
        cid = wid + k * _NW

        def _go():
            for src, dst in gather_copies(b, cid):
                pltpu.make_async_copy(src, dst, sems[b]).wait()

            @plsc.parallel_loop(0, _C, _L, unroll=2)
            def group(o):
                blk = lax.div(o, jnp.int32(_IDX_B))
                ol = o - blk * _IDX_B
                e = lax.iota(jnp.int32, _L) + o
                er = e + _C
                c0 = jnp.zeros((_L,), jnp.int32)
                c1i = jnp.full((_L,), 1, jnp.int32)
                c2i = jnp.full((_L,), 2, jnp.int32)
                rv = rows_v.at[b]
                dx = (plsc.load_gather(rv, [er, c0])
                      - plsc.load_gather(rv, [e, c0])
                      + shx_v[b, pl.ds(o, _L)])
                dy = (plsc.load_gather(rv, [er, c1i])
                      - plsc.load_gather(rv, [e, c1i])
                      + shy_v[b, pl.ds(o, _L)])
                dz = (plsc.load_gather(rv, [er, c2i])
                      - plsc.load_gather(rv, [e, c2i])
                      + shz_v[b, pl.ds(o, _L)])
                len2 = dx * dx + dy * dy + dz * dz
                inv = _rsqrt(len2)
                length = len2 * inv
                len_v[b, pl.ds(o, _L)] = length
                u_v[b, blk, 0, pl.ds(ol, _L)] = dx * inv
                u_v[b, blk, 1, pl.ds(ol, _L)] = dy * inv
                u_v[b, blk, 2, pl.ds(ol, _L)] = dz * inv

                t = jnp.float32(_W1) * lax.min(length, jnp.float32(_CUTOFF))
                u = t - jnp.float32(_PI / 2)
                v = u * u
                s1 = jnp.float32(_COSC[6])
                for cc in (_COSC[5], _COSC[4], _COSC[3], _COSC[2], _COSC[1],
                           _COSC[0]):
                    s1 = s1 * v + jnp.float32(cc)
                cpoly = jnp.float32(_SINC[5])
                for cc in (_SINC[4], _SINC[3], _SINC[2], _SINC[1], _SINC[0]):
                    cpoly = cpoly * v + jnp.float32(cc)
                ct = -(cpoly * u)

                x = length * jnp.float32(1.0 / _CUTOFF)
                x2 = x * x
                x3 = x2 * x
                x6 = x3 * x3
                x7 = x6 * x
                x8 = x7 * x
                env = (jnp.float32(1.0) - jnp.float32(28.0) * x6
                       + jnp.float32(48.0) * x7 - jnp.float32(21.0) * x8)
                env = jnp.where(x < jnp.float32(1.0), env, jnp.float32(0.0))
                scale = jnp.float32(_PREF) * inv * env

                twoc = jnp.float32(2.0) * ct
                sprev = jnp.zeros((_L,), jnp.float32)
                scur = s1
                for n in range(_N_BASES):
                    e_v[b, blk, n, pl.ds(ol, _L)] = scur * scale
                    snext = twoc * scur - sprev
                    sprev, scur = scur, snext

            for src, dst in out_copies(b, cid):
                pltpu.async_copy(src, dst, osems[b])

        drain_outputs(b, cid - 2 * _NW)
        pl.when(cid < _NCHUNKS)(_go)

    start(0, 0)

    def pair(i, carry):
        k = 2 * i
        start(1, k + 1)
        finish(0, k)
        start(0, k + 2)
        finish(1, k + 1)
        return carry

    lax.fori_loop(0, _NK // 2, pair, 0)
    drain_outputs(0, wid + (_NK - 2) * _NW)
    drain_outputs(1, wid + (_NK - 1) * _NW)


_sc_kernel = functools.partial(
    pl.kernel,
    mesh=plsc.VectorSubcoreMesh(core_axis_name="c", subcore_axis_name="s"),
    out_type=(
        jax.ShapeDtypeStruct((_N_EDGES,), jnp.float32),
        jax.ShapeDtypeStruct((_NBLK, 8, _IDX_B), jnp.float32),
        jax.ShapeDtypeStruct((_NBLK, 4, _IDX_B), jnp.float32),
    ),
    scratch_types=[
        pltpu.VMEM((2, 2 * _C // _IDX_B, _IDX_B), jnp.int32),
        pltpu.VMEM((2, 2 * _C, _DPAD), jnp.float32),
        pltpu.VMEM((2, _C), jnp.float32),
        pltpu.VMEM((2, _C), jnp.float32),
        pltpu.VMEM((2, _C), jnp.float32),
        pltpu.VMEM((2, _C), jnp.float32),
        pltpu.VMEM((2, _BPC, 8, _IDX_B), jnp.float32),
        pltpu.VMEM((2, _BPC, 4, _IDX_B), jnp.float32),
        pltpu.SemaphoreType.DMA,
        pltpu.SemaphoreType.DMA,
        pltpu.SemaphoreType.DMA,
        pltpu.SemaphoreType.DMA,
    ],
    compiler_params=pltpu.CompilerParams(
        needs_layout_passes=False, use_tc_tiling_on_sc=False),
)(_sc_body)


@jax.jit
def kernel(positions, edge_index, shifts):
    pos_pad = jnp.concatenate(
        [positions, jnp.zeros((_N_NODES, _DPAD - 3), jnp.float32)], axis=1)
    sender = edge_index[0].reshape(_N_EDGES // _IDX_B, _IDX_B)
    receiver = edge_index[1].reshape(_N_EDGES // _IDX_B, _IDX_B)
    lengths, emb3, unit3 = _sc_kernel(pos_pad, sender, receiver,
                                      shifts[:, 0], shifts[:, 1],
                                      shifts[:, 2])
    emb = emb3.transpose(0, 2, 1).reshape(_N_EDGES, 8)[:, :_N_BASES]
    unit = unit3.transpose(0, 2, 1).reshape(_N_EDGES, 4)[:, :3]
    return (lengths.reshape(_N_EDGES, 1), emb, unit)
